# baseline (speedup 1.0000x reference)
import jax
import jax.numpy as jnp
from jax.experimental import pallas as pl
from jax.experimental.pallas import tpu as pltpu

_EPS = 1e-8


def _simrel_kernel(x_ref, ca_ref, out_hbm, o_vmem, sem):
    b, _, d = x_ref.shape
    ca = ca_ref[...]
    inv_ca = 1.0 / jnp.maximum(jnp.sqrt(jnp.sum(ca * ca, axis=1, keepdims=True)), _EPS)
    ca_bf = ca.astype(jnp.bfloat16)
    ones_row = jnp.ones((1, d), jnp.bfloat16)
    for i in range(b):
        x = x_ref[i]
        ssq = jax.lax.dot_general(
            ones_row, x * x,
            dimension_numbers=(((1,), (1,)), ((), ())),
            preferred_element_type=jnp.float32,
        )
        inv_in = 1.0 / jnp.maximum(jnp.sqrt(ssq), _EPS)
        dots = jax.lax.dot_general(
            ca_bf, x,
            dimension_numbers=(((1,), (1,)), ((), ())),
            preferred_element_type=jnp.float32,
        )
        o_vmem[i] = dots * inv_ca * inv_in
        pltpu.make_async_copy(o_vmem.at[i], out_hbm.at[i], sem.at[i]).start()
    for i in range(b):
        pltpu.make_async_copy(o_vmem.at[i], out_hbm.at[i], sem.at[i]).wait()


def kernel(inputs, labels, class_avgs):
    del labels
    b, s, d = inputs.shape
    k = class_avgs.shape[0]
    x_bf = inputs.astype(jnp.bfloat16)
    out_t = pl.pallas_call(
        _simrel_kernel,
        in_specs=[
            pl.BlockSpec(memory_space=pltpu.MemorySpace.VMEM),
            pl.BlockSpec(memory_space=pltpu.MemorySpace.VMEM),
        ],
        out_specs=pl.BlockSpec(memory_space=pltpu.MemorySpace.HBM),
        out_shape=jax.ShapeDtypeStruct((b, k, s), jnp.float32),
        scratch_shapes=[
            pltpu.VMEM((b, k, s), jnp.float32),
            pltpu.SemaphoreType.DMA((b,)),
        ],
        compiler_params=pltpu.CompilerParams(allow_input_fusion=[True, False]),
    )(x_bf, class_avgs)
    return jnp.swapaxes(out_t, 1, 2)


# speedup vs baseline: 1.0603x; 1.0603x over previous
import jax
import jax.numpy as jnp
from jax.experimental import pallas as pl

_EPS = 1e-8


def _simrel_kernel(x_ref, ca_ref, out_ref):
    b = x_ref.shape[0]
    ca = ca_ref[...]
    inv_ca = 1.0 / jnp.maximum(jnp.sqrt(jnp.sum(ca * ca, axis=1, keepdims=True)), _EPS)
    for i in range(b):
        x = x_ref[i]
        inv_in = 1.0 / jnp.maximum(jnp.sqrt(jnp.sum(x * x, axis=1)), _EPS)
        dots = jax.lax.dot_general(
            ca, x,
            dimension_numbers=(((1,), (1,)), ((), ())),
            preferred_element_type=jnp.float32,
        )
        out_ref[i] = dots * inv_ca * inv_in[None, :]


def kernel(inputs, labels, class_avgs):
    del labels
    b, s, d = inputs.shape
    k = class_avgs.shape[0]
    out_t = pl.pallas_call(
        _simrel_kernel,
        out_shape=jax.ShapeDtypeStruct((b, k, s), jnp.float32),
    )(inputs, class_avgs)
    return jnp.swapaxes(out_t, 1, 2)
